# Initial kernel scaffold; baseline (speedup 1.0000x reference)
import jax
import jax.numpy as jnp
from jax.experimental import pallas as pl


def kernel(z, embedding):
    raise NotImplementedError("write your pallas kernel here")


# speedup vs baseline: 1.2759x; 1.2759x over previous
import functools

import jax
import jax.numpy as jnp
from jax import lax
from jax.experimental import pallas as pl
from jax.experimental.pallas import tpu as pltpu
from jax.experimental.pallas import tpu_sc as plsc

CODEBOOK = 8192
DIM = 256
TOKENS = 16384
BETA = 0.25

BM = 512
BN = 2048
NI = TOKENS // BM
NJ = CODEBOOK // BN


def _dist_argmin_body(zn_ref, en_ref, z_ref, e_ref, idx_ref, loss_ref, acc_ref):
    i = pl.program_id(0)
    z = z_ref[...]
    zn = zn_ref[...]

    best_v = None
    best_i = None
    for j in range(NJ):
        e = e_ref[pl.ds(j * BN, BN), :]
        en = en_ref[:, pl.ds(j * BN, BN)]
        mm = lax.dot_general(z, e, (((1,), (1,)), ((), ())),
                             preferred_element_type=jnp.float32)
        d = (zn + en) - 2.0 * mm
        tmin = jnp.min(d, axis=1, keepdims=True)
        iota = lax.broadcasted_iota(jnp.int32, (BM, BN), 1)
        cand = jnp.where(d == tmin, iota, BN)
        targ = jnp.min(cand, axis=1, keepdims=True) + (j * BN)
        if best_v is None:
            best_v, best_i = tmin, targ
        else:
            better = tmin < best_v
            best_v = jnp.where(better, tmin, best_v)
            best_i = jnp.where(better, targ, best_i)

    idx_ref[...] = best_i
    part = jnp.sum(best_v)

    @pl.when(i == 0)
    def _():
        acc_ref[0] = part

    @pl.when(i > 0)
    def _():
        acc_ref[0] = acc_ref[0] + part

    @pl.when(i == NI - 1)
    def _():
        loss_ref[0] = acc_ref[0] * ((1.0 + BETA) / (TOKENS * DIM))


def _distance_argmin(zn, en, z_flat, embedding, interpret=False):
    return pl.pallas_call(
        _dist_argmin_body,
        grid=(NI,),
        in_specs=[
            pl.BlockSpec((BM, 1), lambda i: (i, 0)),
            pl.BlockSpec((1, CODEBOOK), lambda i: (0, 0)),
            pl.BlockSpec((BM, DIM), lambda i: (i, 0)),
            pl.BlockSpec((CODEBOOK, DIM), lambda i: (0, 0)),
        ],
        out_specs=[
            pl.BlockSpec((BM, 1), lambda i: (i, 0)),
            pl.BlockSpec(memory_space=pltpu.SMEM),
        ],
        out_shape=[
            jax.ShapeDtypeStruct((TOKENS, 1), jnp.int32),
            jax.ShapeDtypeStruct((1,), jnp.float32),
        ],
        scratch_shapes=[pltpu.SMEM((1,), jnp.float32)],
        interpret=interpret,
    )(zn, en, z_flat, embedding)


def _sc_gather(embedding, idx):
    info = plsc.get_sparse_core_info()
    nc, ns = info.num_cores, info.num_subcores
    nw = nc * ns
    b_per_w = TOKENS // nw
    chunk = 256
    nch = b_per_w // chunk
    mesh = plsc.VectorSubcoreMesh(core_axis_name="c", subcore_axis_name="s")

    @functools.partial(
        pl.kernel,
        mesh=mesh,
        out_type=jax.ShapeDtypeStruct((TOKENS, DIM), jnp.float32),
        scratch_types=[
            pltpu.VMEM((b_per_w,), jnp.int32),
            pltpu.VMEM((chunk, DIM), jnp.float32),
            pltpu.SemaphoreType.DMA,
        ],
    )
    def gather_kernel(table_hbm, idx_hbm, out_hbm, idx_v, rows_v, sem):
        wid = lax.axis_index("s") * nc + lax.axis_index("c")
        base = wid * b_per_w
        pltpu.sync_copy(idx_hbm.at[pl.ds(base, b_per_w)], idx_v)
        for c in range(nch):
            pltpu.async_copy(
                table_hbm.at[idx_v.at[pl.ds(c * chunk, chunk)]], rows_v, sem
            ).wait()
            pltpu.sync_copy(rows_v, out_hbm.at[pl.ds(base + c * chunk, chunk)])

    return gather_kernel(embedding, idx)


def kernel(z, embedding):
    zp = jnp.transpose(z, (0, 2, 3, 1))
    z_flat = zp.reshape(-1, DIM)
    zn = jnp.sum(z_flat ** 2, axis=1, keepdims=True)
    en = jnp.sum(embedding ** 2, axis=1).reshape(1, CODEBOOK)

    idx2d, loss_arr = _distance_argmin(zn, en, z_flat, embedding)
    idx = idx2d.reshape(TOKENS)

    zq_flat = _sc_gather(embedding, idx)

    z_q_out = jnp.transpose(zq_flat.reshape(zp.shape), (0, 3, 1, 2))
    return (z_q_out, loss_arr[0], (None, None, idx))
